# baseline (speedup 1.0000x reference)
import functools

import jax
import jax.numpy as jnp
from jax import lax
from jax.experimental import pallas as pl
from jax.experimental.pallas import tpu as pltpu
from jax.experimental.pallas import tpu_sc as plsc

BATCH = 16384
NUM_SKILLS = 100000
SKILL_DIM = 64
HIDDEN_DIM = 128
LIST_LEN = 20

NUM_CORES = 2
NUM_SUBCORES = 16
NW = NUM_CORES * NUM_SUBCORES

CHUNK = 16
IDS_PER_CHUNK = CHUNK * 2 * LIST_LEN
IDX_ROWS = IDS_PER_CHUNK // 128
CHUNKS_PER_W = BATCH // (NW * CHUNK)
TOTAL_CHUNKS = BATCH // CHUNK


def _pooling_sc(table, ids1, ids2):
    mesh = plsc.VectorSubcoreMesh(
        core_axis_name="c", subcore_axis_name="s",
        num_cores=NUM_CORES, num_subcores=NUM_SUBCORES)

    @functools.partial(
        pl.kernel,
        out_type=jax.ShapeDtypeStruct((BATCH, 2 * SKILL_DIM), jnp.float32),
        mesh=mesh,
        scratch_types=[
            pltpu.VMEM((CHUNK, LIST_LEN), jnp.int32),
            pltpu.VMEM((CHUNK, LIST_LEN), jnp.int32),
            pltpu.VMEM((CHUNK, LIST_LEN), jnp.int32),
            pltpu.VMEM((CHUNK, LIST_LEN), jnp.int32),
            pltpu.VMEM((IDS_PER_CHUNK, SKILL_DIM), jnp.float32),
            pltpu.VMEM((IDS_PER_CHUNK, SKILL_DIM), jnp.float32),
            pltpu.VMEM((CHUNK, 2 * SKILL_DIM), jnp.float32),
            pltpu.VMEM((CHUNK, 2 * SKILL_DIM), jnp.float32),
            pltpu.SemaphoreType.DMA,
            pltpu.SemaphoreType.DMA,
        ],
        compiler_params=pltpu.CompilerParams(use_tc_tiling_on_sc=False),
    )
    def k(table_hbm, ids1_hbm, ids2_hbm, out_hbm,
          idx1_a, idx2_a, idx1_b, idx2_b, rows_a, rows_b, out_a, out_b,
          sem_a, sem_b):
        wid = lax.axis_index("s") * NUM_CORES + lax.axis_index("c")
        base = wid * CHUNKS_PER_W

        def fire(kk, idx1_v, idx2_v, rows_v, sem):
            eb = kk * CHUNK
            pltpu.sync_copy(ids1_hbm.at[pl.ds(eb, CHUNK)], idx1_v)
            pltpu.sync_copy(ids2_hbm.at[pl.ds(eb, CHUNK)], idx2_v)
            for i in range(CHUNK):
                pltpu.async_copy(
                    table_hbm.at[idx1_v.at[i]],
                    rows_v.at[pl.ds(i * 2 * LIST_LEN, LIST_LEN)], sem)
                pltpu.async_copy(
                    table_hbm.at[idx2_v.at[i]],
                    rows_v.at[pl.ds(i * 2 * LIST_LEN + LIST_LEN, LIST_LEN)],
                    sem)

        def drain(idx1_v, idx2_v, rows_v, sem):
            for i in range(CHUNK):
                pltpu.make_async_copy(
                    table_hbm.at[idx1_v.at[i]],
                    rows_v.at[pl.ds(i * 2 * LIST_LEN, LIST_LEN)], sem).wait()
                pltpu.make_async_copy(
                    table_hbm.at[idx2_v.at[i]],
                    rows_v.at[pl.ds(i * 2 * LIST_LEN + LIST_LEN, LIST_LEN)],
                    sem).wait()

        def reduce_store(kk, rows_v, out_v):
            @pl.loop(0, CHUNK)
            def _elem(i):
                row0 = i * (2 * LIST_LEN)
                for half in range(2):
                    rbase = row0 + half * LIST_LEN
                    for d in range(SKILL_DIM // 16):
                        acc = rows_v[rbase, pl.ds(d * 16, 16)]
                        for r in range(1, LIST_LEN):
                            acc = acc + rows_v[rbase + r, pl.ds(d * 16, 16)]
                        out_v[i, pl.ds(half * SKILL_DIM + d * 16, 16)] = acc

            pltpu.sync_copy(out_v, out_hbm.at[pl.ds(kk * CHUNK, CHUNK)])

        fire(base, idx1_a, idx2_a, rows_a, sem_a)

        @pl.loop(0, CHUNKS_PER_W // 2)
        def _pair(c2):
            c0 = base + 2 * c2
            fire(c0 + 1, idx1_b, idx2_b, rows_b, sem_b)
            drain(idx1_a, idx2_a, rows_a, sem_a)
            reduce_store(c0, rows_a, out_a)

            @pl.when(c2 < CHUNKS_PER_W // 2 - 1)
            def _():
                fire(c0 + 2, idx1_a, idx2_a, rows_a, sem_a)

            drain(idx1_b, idx2_b, rows_b, sem_b)
            reduce_store(c0 + 1, rows_b, out_b)

    return k(table, ids1, ids2)


def _mlp_body(x_ref, w1t_ref, b1_ref, w2_ref, b2_ref, o_ref):
    h = jnp.dot(x_ref[...], w1t_ref[...], preferred_element_type=jnp.float32)
    h = jnp.maximum(h + b1_ref[...], 0.0)
    z = jnp.sum(h * w2_ref[...], axis=1, keepdims=True) + b2_ref[...]
    o_ref[...] = 1.0 / (1.0 + jnp.exp(-z))


def _mlp_tc(x, w1t, b1, w2, b2):
    BM = 1024
    grid = (BATCH // BM,)
    return pl.pallas_call(
        _mlp_body,
        grid=grid,
        in_specs=[
            pl.BlockSpec((BM, 2 * SKILL_DIM), lambda i: (i, 0)),
            pl.BlockSpec((2 * SKILL_DIM, HIDDEN_DIM), lambda i: (0, 0)),
            pl.BlockSpec((1, HIDDEN_DIM), lambda i: (0, 0)),
            pl.BlockSpec((1, HIDDEN_DIM), lambda i: (0, 0)),
            pl.BlockSpec((1, 1), lambda i: (0, 0)),
        ],
        out_specs=pl.BlockSpec((BM, 1), lambda i: (i, 0)),
        out_shape=jax.ShapeDtypeStruct((BATCH, 1), jnp.float32),
    )(x, w1t, b1, w2, b2)


def kernel(skill_ids_1, skill_ids_2, table, W1, b1, W2, b2):
    pooled = _pooling_sc(table,
                         skill_ids_1.astype(jnp.int32),
                         skill_ids_2.astype(jnp.int32))
    w1t = W1.T * (1.0 / LIST_LEN)
    return _mlp_tc(pooled, w1t, b1.reshape(1, -1), W2, b2.reshape(1, 1))


# speedup vs baseline: 1.0274x; 1.0274x over previous
import functools

import jax
import jax.numpy as jnp
from jax import lax
from jax.experimental import pallas as pl
from jax.experimental.pallas import tpu as pltpu
from jax.experimental.pallas import tpu_sc as plsc

BATCH = 16384
NUM_SKILLS = 100000
SKILL_DIM = 64
HIDDEN_DIM = 128
LIST_LEN = 20

NUM_CORES = 2
NUM_SUBCORES = 16
NW = NUM_CORES * NUM_SUBCORES

CHUNK = 16
IDS_PER_CHUNK = CHUNK * 2 * LIST_LEN
IDX_ROWS = IDS_PER_CHUNK // 128
CHUNKS_PER_W = BATCH // (NW * CHUNK)
TOTAL_CHUNKS = BATCH // CHUNK


def _pooling_sc(table, ids1, ids2):
    mesh = plsc.VectorSubcoreMesh(
        core_axis_name="c", subcore_axis_name="s",
        num_cores=NUM_CORES, num_subcores=NUM_SUBCORES)

    @functools.partial(
        pl.kernel,
        out_type=jax.ShapeDtypeStruct((BATCH, 2 * SKILL_DIM), jnp.float32),
        mesh=mesh,
        scratch_types=[
            pltpu.VMEM((2 * CHUNK, 128), jnp.int32),
            pltpu.VMEM((2 * CHUNK, 128), jnp.int32),
            pltpu.VMEM((IDX_ROWS, 128), jnp.int32),
            pltpu.VMEM((IDX_ROWS, 128), jnp.int32),
            pltpu.VMEM((IDS_PER_CHUNK // 16, 16), jnp.int32),
            pltpu.VMEM((IDS_PER_CHUNK // 16, 16), jnp.int32),
            pltpu.VMEM((IDS_PER_CHUNK, SKILL_DIM), jnp.float32),
            pltpu.VMEM((IDS_PER_CHUNK, SKILL_DIM), jnp.float32),
            pltpu.VMEM((CHUNK, 2 * SKILL_DIM), jnp.float32),
            pltpu.VMEM((CHUNK, 2 * SKILL_DIM), jnp.float32),
            pltpu.SemaphoreType.DMA,
            pltpu.SemaphoreType.DMA,
        ],
        compiler_params=pltpu.CompilerParams(
            use_tc_tiling_on_sc=False, needs_layout_passes=False),
    )
    def k(table_hbm, ids1_hbm, ids2_hbm, out_hbm,
          raw_a, raw_b, pk_a, pk_b, prow, pcol,
          rows_a, rows_b, out_a, out_b, sem_a, sem_b):
        wid = lax.axis_index("s") * NUM_CORES + lax.axis_index("c")
        base = wid * CHUNKS_PER_W
        NVEC = IDS_PER_CHUNK // 16

        lane = lax.iota(jnp.int32, 16)
        for g in range(NVEC):
            lo = 16 * g
            i0 = lo // (2 * LIST_LEN)
            t = (2 * LIST_LEN) * (i0 + 1) - lo
            i = jnp.where(lane >= t, i0 + 1, i0)
            rem = lo + lane - (2 * LIST_LEN) * i
            h = jnp.where(rem >= LIST_LEN, 1, 0)
            r = rem - LIST_LEN * h
            prow[g] = i + CHUNK * h
            pcol[g] = r

        def fire(kk, raw_v, pk_v, rows_v, sem):
            eb = kk * CHUNK
            pltpu.sync_copy(ids1_hbm.at[pl.ds(eb, CHUNK)],
                            raw_v.at[pl.ds(0, CHUNK)])
            pltpu.sync_copy(ids2_hbm.at[pl.ds(eb, CHUNK)],
                            raw_v.at[pl.ds(CHUNK, CHUNK)])
            for g in range(NVEC):
                pk_v[g // 8, pl.ds((g % 8) * 16, 16)] = plsc.load_gather(
                    raw_v, [prow[g], pcol[g]])
            for j in range(IDX_ROWS):
                pltpu.async_copy(
                    table_hbm.at[pk_v.at[j]],
                    rows_v.at[pl.ds(j * 128, 128)], sem)

        def drain(pk_v, rows_v, sem):
            for j in range(IDX_ROWS):
                pltpu.make_async_copy(
                    table_hbm.at[pk_v.at[j]],
                    rows_v.at[pl.ds(j * 128, 128)], sem).wait()

        def reduce_store(kk, rows_v, out_v):
            @pl.loop(0, CHUNK)
            def _elem(i):
                row0 = i * (2 * LIST_LEN)
                for half in range(2):
                    rbase = row0 + half * LIST_LEN
                    for d in range(SKILL_DIM // 16):
                        acc = rows_v[rbase, pl.ds(d * 16, 16)]
                        for r in range(1, LIST_LEN):
                            acc = acc + rows_v[rbase + r, pl.ds(d * 16, 16)]
                        out_v[i, pl.ds(half * SKILL_DIM + d * 16, 16)] = acc

            pltpu.sync_copy(out_v, out_hbm.at[pl.ds(kk * CHUNK, CHUNK)])

        fire(base, raw_a, pk_a, rows_a, sem_a)

        @pl.loop(0, CHUNKS_PER_W // 2)
        def _pair(c2):
            c0 = base + 2 * c2
            fire(c0 + 1, raw_b, pk_b, rows_b, sem_b)
            drain(pk_a, rows_a, sem_a)
            reduce_store(c0, rows_a, out_a)

            @pl.when(c2 < CHUNKS_PER_W // 2 - 1)
            def _():
                fire(c0 + 2, raw_a, pk_a, rows_a, sem_a)

            drain(pk_b, rows_b, sem_b)
            reduce_store(c0 + 1, rows_b, out_b)

    return k(table, ids1, ids2)


def _mlp_body(x_ref, w1t_ref, b1_ref, w2_ref, b2_ref, o_ref):
    h = jnp.dot(x_ref[...], w1t_ref[...], preferred_element_type=jnp.float32)
    h = jnp.maximum(h + b1_ref[...], 0.0)
    z = jnp.sum(h * w2_ref[...], axis=1, keepdims=True) + b2_ref[...]
    o_ref[...] = 1.0 / (1.0 + jnp.exp(-z))


def _mlp_tc(x, w1t, b1, w2, b2):
    BM = 1024
    grid = (BATCH // BM,)
    return pl.pallas_call(
        _mlp_body,
        grid=grid,
        in_specs=[
            pl.BlockSpec((BM, 2 * SKILL_DIM), lambda i: (i, 0)),
            pl.BlockSpec((2 * SKILL_DIM, HIDDEN_DIM), lambda i: (0, 0)),
            pl.BlockSpec((1, HIDDEN_DIM), lambda i: (0, 0)),
            pl.BlockSpec((1, HIDDEN_DIM), lambda i: (0, 0)),
            pl.BlockSpec((1, 1), lambda i: (0, 0)),
        ],
        out_specs=pl.BlockSpec((BM, 1), lambda i: (i, 0)),
        out_shape=jax.ShapeDtypeStruct((BATCH, 1), jnp.float32),
    )(x, w1t, b1, w2, b2)


def kernel(skill_ids_1, skill_ids_2, table, W1, b1, W2, b2):
    pad = ((0, 0), (0, 128 - LIST_LEN))
    ids1p = jnp.pad(skill_ids_1.astype(jnp.int32), pad)
    ids2p = jnp.pad(skill_ids_2.astype(jnp.int32), pad)
    pooled = _pooling_sc(table, ids1p, ids2p)
    w1t = W1.T * (1.0 / LIST_LEN)
    return _mlp_tc(pooled, w1t, b1.reshape(1, -1), W2, b2.reshape(1, 1))


# speedup vs baseline: 1.1596x; 1.1286x over previous
import functools

import jax
import jax.numpy as jnp
from jax import lax
from jax.experimental import pallas as pl
from jax.experimental.pallas import tpu as pltpu
from jax.experimental.pallas import tpu_sc as plsc

BATCH = 16384
NUM_SKILLS = 100000
SKILL_DIM = 64
HIDDEN_DIM = 128
LIST_LEN = 20

NUM_CORES = 2
NUM_SUBCORES = 16
NW = NUM_CORES * NUM_SUBCORES

CHUNK = 32
IDS_PER_UNIT = CHUNK * LIST_LEN
UNIT_ROWS = IDS_PER_UNIT // 128
CHUNKS_PER_W = BATCH // (NW * CHUNK)
TOTAL_CHUNKS = BATCH // CHUNK


def _pooling_sc(table, idx1, idx2):
    mesh = plsc.VectorSubcoreMesh(
        core_axis_name="c", subcore_axis_name="s",
        num_cores=NUM_CORES, num_subcores=NUM_SUBCORES)

    @functools.partial(
        pl.kernel,
        out_type=jax.ShapeDtypeStruct((BATCH, 2 * SKILL_DIM), jnp.float32),
        mesh=mesh,
        scratch_types=[
            pltpu.VMEM((UNIT_ROWS, 128), jnp.int32),
            pltpu.VMEM((UNIT_ROWS, 128), jnp.int32),
            pltpu.VMEM((IDS_PER_UNIT, SKILL_DIM), jnp.float32),
            pltpu.VMEM((IDS_PER_UNIT, SKILL_DIM), jnp.float32),
            pltpu.VMEM((CHUNK, 2 * SKILL_DIM), jnp.float32),
            pltpu.SemaphoreType.DMA,
            pltpu.SemaphoreType.DMA,
        ],
        compiler_params=pltpu.CompilerParams(use_tc_tiling_on_sc=False),
    )
    def k(table_hbm, idx1_hbm, idx2_hbm, out_hbm,
          idx_a, idx_b, rows_a, rows_b, out_v, sem_a, sem_b):
        wid = lax.axis_index("s") * NUM_CORES + lax.axis_index("c")
        base = wid * CHUNKS_PER_W

        def fire(kk, src_hbm, idx_v, rows_v, sem):
            pltpu.sync_copy(src_hbm.at[kk], idx_v)
            for j in range(UNIT_ROWS):
                pltpu.async_copy(
                    table_hbm.at[idx_v.at[j]],
                    rows_v.at[pl.ds(j * 128, 128)], sem)

        def drain(idx_v, rows_v, sem):
            for j in range(UNIT_ROWS):
                pltpu.make_async_copy(
                    table_hbm.at[idx_v.at[j]],
                    rows_v.at[pl.ds(j * 128, 128)], sem).wait()

        def reduce(rows_v, half):
            @pl.loop(0, CHUNK)
            def _elem(i):
                rbase = i * LIST_LEN
                for d in range(SKILL_DIM // 16):
                    acc = rows_v[rbase, pl.ds(d * 16, 16)]
                    for r in range(1, LIST_LEN):
                        acc = acc + rows_v[rbase + r, pl.ds(d * 16, 16)]
                    out_v[i, pl.ds(half * SKILL_DIM + d * 16, 16)] = acc

        fire(base, idx1_hbm, idx_a, rows_a, sem_a)

        @pl.loop(0, CHUNKS_PER_W)
        def _chunk(c):
            kk = base + c
            fire(kk, idx2_hbm, idx_b, rows_b, sem_b)
            drain(idx_a, rows_a, sem_a)
            reduce(rows_a, 0)

            @pl.when(c < CHUNKS_PER_W - 1)
            def _():
                fire(kk + 1, idx1_hbm, idx_a, rows_a, sem_a)

            drain(idx_b, rows_b, sem_b)
            reduce(rows_b, 1)
            pltpu.sync_copy(out_v, out_hbm.at[pl.ds(kk * CHUNK, CHUNK)])

    return k(table, idx1, idx2)


def _mlp_body(x_ref, w1t_ref, b1_ref, w2_ref, b2_ref, o_ref):
    h = jnp.dot(x_ref[...], w1t_ref[...], preferred_element_type=jnp.float32)
    h = jnp.maximum(h + b1_ref[...], 0.0)
    z = jnp.sum(h * w2_ref[...], axis=1, keepdims=True) + b2_ref[...]
    o_ref[...] = 1.0 / (1.0 + jnp.exp(-z))


def _mlp_tc(x, w1t, b1, w2, b2):
    BM = 1024
    grid = (BATCH // BM,)
    return pl.pallas_call(
        _mlp_body,
        grid=grid,
        in_specs=[
            pl.BlockSpec((BM, 2 * SKILL_DIM), lambda i: (i, 0)),
            pl.BlockSpec((2 * SKILL_DIM, HIDDEN_DIM), lambda i: (0, 0)),
            pl.BlockSpec((1, HIDDEN_DIM), lambda i: (0, 0)),
            pl.BlockSpec((1, HIDDEN_DIM), lambda i: (0, 0)),
            pl.BlockSpec((1, 1), lambda i: (0, 0)),
        ],
        out_specs=pl.BlockSpec((BM, 1), lambda i: (i, 0)),
        out_shape=jax.ShapeDtypeStruct((BATCH, 1), jnp.float32),
    )(x, w1t, b1, w2, b2)


def kernel(skill_ids_1, skill_ids_2, table, W1, b1, W2, b2):
    idx1 = skill_ids_1.astype(jnp.int32).reshape(TOTAL_CHUNKS, UNIT_ROWS, 128)
    idx2 = skill_ids_2.astype(jnp.int32).reshape(TOTAL_CHUNKS, UNIT_ROWS, 128)
    pooled = _pooling_sc(table, idx1, idx2)
    w1t = W1.T * (1.0 / LIST_LEN)
    return _mlp_tc(pooled, w1t, b1.reshape(1, -1), W2, b2.reshape(1, 1))


# speedup vs baseline: 1.2239x; 1.0554x over previous
import functools

import jax
import jax.numpy as jnp
from jax import lax
from jax.experimental import pallas as pl
from jax.experimental.pallas import tpu as pltpu
from jax.experimental.pallas import tpu_sc as plsc

BATCH = 16384
NUM_SKILLS = 100000
SKILL_DIM = 64
HIDDEN_DIM = 128
LIST_LEN = 20

NUM_CORES = 2
NUM_SUBCORES = 16
NW = NUM_CORES * NUM_SUBCORES

CHUNK = 32
IDS_PER_UNIT = CHUNK * LIST_LEN
UNIT_ROWS = IDS_PER_UNIT // 128
CHUNKS_PER_W = BATCH // (NW * CHUNK)
TOTAL_CHUNKS = BATCH // CHUNK


def _pooling_sc(table, idx1, idx2, nbatch):
    chunks_per_w = nbatch // (NW * CHUNK)
    mesh = plsc.VectorSubcoreMesh(
        core_axis_name="c", subcore_axis_name="s",
        num_cores=NUM_CORES, num_subcores=NUM_SUBCORES)

    @functools.partial(
        pl.kernel,
        out_type=jax.ShapeDtypeStruct((nbatch, 2 * SKILL_DIM), jnp.float32),
        mesh=mesh,
        scratch_types=[
            pltpu.VMEM((UNIT_ROWS, 128), jnp.int32),
            pltpu.VMEM((UNIT_ROWS, 128), jnp.int32),
            pltpu.VMEM((IDS_PER_UNIT, SKILL_DIM), jnp.float32),
            pltpu.VMEM((IDS_PER_UNIT, SKILL_DIM), jnp.float32),
            pltpu.VMEM((CHUNK, 2 * SKILL_DIM), jnp.float32),
            pltpu.SemaphoreType.DMA,
            pltpu.SemaphoreType.DMA,
        ],
        compiler_params=pltpu.CompilerParams(use_tc_tiling_on_sc=False),
    )
    def k(table_hbm, idx1_hbm, idx2_hbm, out_hbm,
          idx_a, idx_b, rows_a, rows_b, out_v, sem_a, sem_b):
        wid = lax.axis_index("s") * NUM_CORES + lax.axis_index("c")
        base = wid * chunks_per_w

        def fire(kk, src_hbm, idx_v, rows_v, sem):
            pltpu.sync_copy(src_hbm.at[kk], idx_v)
            for j in range(UNIT_ROWS):
                pltpu.async_copy(
                    table_hbm.at[idx_v.at[j]],
                    rows_v.at[pl.ds(j * 128, 128)], sem)

        def drain(idx_v, rows_v, sem):
            for j in range(UNIT_ROWS):
                pltpu.make_async_copy(
                    table_hbm.at[idx_v.at[j]],
                    rows_v.at[pl.ds(j * 128, 128)], sem).wait()

        def reduce(rows_v, half):
            @pl.loop(0, CHUNK)
            def _elem(i):
                rbase = i * LIST_LEN
                for d in range(SKILL_DIM // 16):
                    acc = rows_v[rbase, pl.ds(d * 16, 16)]
                    for r in range(1, LIST_LEN):
                        acc = acc + rows_v[rbase + r, pl.ds(d * 16, 16)]
                    out_v[i, pl.ds(half * SKILL_DIM + d * 16, 16)] = acc

        fire(base, idx1_hbm, idx_a, rows_a, sem_a)

        @pl.loop(0, chunks_per_w)
        def _chunk(c):
            kk = base + c
            fire(kk, idx2_hbm, idx_b, rows_b, sem_b)
            drain(idx_a, rows_a, sem_a)
            reduce(rows_a, 0)

            @pl.when(c < chunks_per_w - 1)
            def _():
                fire(kk + 1, idx1_hbm, idx_a, rows_a, sem_a)

            drain(idx_b, rows_b, sem_b)
            reduce(rows_b, 1)
            pltpu.sync_copy(out_v, out_hbm.at[pl.ds(kk * CHUNK, CHUNK)])

    return k(table, idx1, idx2)


def _mlp_body(x_ref, w1t_ref, b1_ref, w2_ref, b2_ref, o_ref):
    h = jnp.dot(x_ref[...], w1t_ref[...], preferred_element_type=jnp.float32)
    h = jnp.maximum(h + b1_ref[...], 0.0)
    z = jnp.sum(h * w2_ref[...], axis=1, keepdims=True) + b2_ref[...]
    o_ref[...] = 1.0 / (1.0 + jnp.exp(-z))


def _mlp_tc(x, w1t, b1, w2, b2):
    BM = 1024
    nbatch = x.shape[0]
    grid = (nbatch // BM,)
    return pl.pallas_call(
        _mlp_body,
        grid=grid,
        in_specs=[
            pl.BlockSpec((BM, 2 * SKILL_DIM), lambda i: (i, 0)),
            pl.BlockSpec((2 * SKILL_DIM, HIDDEN_DIM), lambda i: (0, 0)),
            pl.BlockSpec((1, HIDDEN_DIM), lambda i: (0, 0)),
            pl.BlockSpec((1, HIDDEN_DIM), lambda i: (0, 0)),
            pl.BlockSpec((1, 1), lambda i: (0, 0)),
        ],
        out_specs=pl.BlockSpec((BM, 1), lambda i: (i, 0)),
        out_shape=jax.ShapeDtypeStruct((nbatch, 1), jnp.float32),
    )(x, w1t, b1, w2, b2)


def kernel(skill_ids_1, skill_ids_2, table, W1, b1, W2, b2):
    ids1 = skill_ids_1.astype(jnp.int32)
    ids2 = skill_ids_2.astype(jnp.int32)
    w1t = W1.T * (1.0 / LIST_LEN)
    b1r = b1.reshape(1, -1)
    b2r = b2.reshape(1, 1)
    h = BATCH // 2
    hc = h // CHUNK
    outs = []
    for s in range(2):
        i1 = ids1[s * h:(s + 1) * h].reshape(hc, UNIT_ROWS, 128)
        i2 = ids2[s * h:(s + 1) * h].reshape(hc, UNIT_ROWS, 128)
        pooled = _pooling_sc(table, i1, i2, h)
        outs.append(_mlp_tc(pooled, w1t, b1r, W2, b2r))
    return jnp.concatenate(outs, axis=0)


# speedup vs baseline: 1.3172x; 1.0762x over previous
import functools

import jax
import jax.numpy as jnp
from jax import lax
from jax.experimental import pallas as pl
from jax.experimental.pallas import tpu as pltpu
from jax.experimental.pallas import tpu_sc as plsc

BATCH = 16384
NUM_SKILLS = 100000
SKILL_DIM = 64
HIDDEN_DIM = 128
LIST_LEN = 20

NUM_CORES = 2
NUM_SUBCORES = 16
NW = NUM_CORES * NUM_SUBCORES

CHUNK = 32
IDS_PER_UNIT = CHUNK * LIST_LEN
UNIT_ROWS = IDS_PER_UNIT // 128
CHUNKS_PER_W = BATCH // (NW * CHUNK)
TOTAL_CHUNKS = BATCH // CHUNK


def _pooling_sc(table, idx1, idx2, nbatch):
    chunks_per_w = nbatch // (NW * CHUNK)
    mesh = plsc.VectorSubcoreMesh(
        core_axis_name="c", subcore_axis_name="s",
        num_cores=NUM_CORES, num_subcores=NUM_SUBCORES)

    @functools.partial(
        pl.kernel,
        out_type=jax.ShapeDtypeStruct((nbatch, 2 * SKILL_DIM), jnp.float32),
        mesh=mesh,
        scratch_types=[
            pltpu.VMEM((2, UNIT_ROWS, 128), jnp.int32),
            pltpu.VMEM((2, UNIT_ROWS, 128), jnp.int32),
            pltpu.VMEM((IDS_PER_UNIT, SKILL_DIM), jnp.float32),
            pltpu.VMEM((IDS_PER_UNIT, SKILL_DIM), jnp.float32),
            pltpu.VMEM((CHUNK, 2 * SKILL_DIM), jnp.float32),
            pltpu.SemaphoreType.DMA,
            pltpu.SemaphoreType.DMA,
            pltpu.SemaphoreType.DMA,
            pltpu.SemaphoreType.DMA,
        ],
        compiler_params=pltpu.CompilerParams(use_tc_tiling_on_sc=False),
    )
    def k(table_hbm, idx1_hbm, idx2_hbm, out_hbm,
          idx_a, idx_b, rows_a, rows_b, out_v,
          sem_a, sem_b, isem_a, isem_b):
        wid = lax.axis_index("s") * NUM_CORES + lax.axis_index("c")
        base = wid * chunks_per_w

        def idx_copy(kk, src_hbm, idx_v, p, isem):
            pltpu.async_copy(src_hbm.at[kk], idx_v.at[p], isem)

        def fire(kk, src_hbm, idx_v, p, isem, rows_v, sem):
            pltpu.make_async_copy(src_hbm.at[kk], idx_v.at[p], isem).wait()
            for j in range(UNIT_ROWS):
                pltpu.async_copy(
                    table_hbm.at[idx_v.at[p, j]],
                    rows_v.at[pl.ds(j * 128, 128)], sem)

        def drain(idx_v, p, rows_v, sem):
            for j in range(UNIT_ROWS):
                pltpu.make_async_copy(
                    table_hbm.at[idx_v.at[p, j]],
                    rows_v.at[pl.ds(j * 128, 128)], sem).wait()

        def reduce(rows_v, half):
            @pl.loop(0, CHUNK)
            def _elem(i):
                rbase = i * LIST_LEN
                for d in range(SKILL_DIM // 16):
                    acc = rows_v[rbase, pl.ds(d * 16, 16)]
                    for r in range(1, LIST_LEN):
                        acc = acc + rows_v[rbase + r, pl.ds(d * 16, 16)]
                    out_v[i, pl.ds(half * SKILL_DIM + d * 16, 16)] = acc

        idx_copy(base, idx1_hbm, idx_a, 0, isem_a)
        idx_copy(base, idx2_hbm, idx_b, 0, isem_b)
        idx_copy(base + 1, idx1_hbm, idx_a, 1, isem_a)
        idx_copy(base + 1, idx2_hbm, idx_b, 1, isem_b)
        fire(base, idx1_hbm, idx_a, 0, isem_a, rows_a, sem_a)

        @pl.loop(0, chunks_per_w)
        def _chunk(c):
            kk = base + c
            p = c % 2
            fire(kk, idx2_hbm, idx_b, p, isem_b, rows_b, sem_b)
            drain(idx_a, p, rows_a, sem_a)

            @pl.when(c < chunks_per_w - 2)
            def _():
                idx_copy(kk + 2, idx1_hbm, idx_a, p, isem_a)

            reduce(rows_a, 0)

            @pl.when(c < chunks_per_w - 1)
            def _():
                fire(kk + 1, idx1_hbm, idx_a, 1 - p, isem_a, rows_a, sem_a)

            drain(idx_b, p, rows_b, sem_b)

            @pl.when(c < chunks_per_w - 2)
            def _():
                idx_copy(kk + 2, idx2_hbm, idx_b, p, isem_b)

            reduce(rows_b, 1)
            pltpu.sync_copy(out_v, out_hbm.at[pl.ds(kk * CHUNK, CHUNK)])

    return k(table, idx1, idx2)


def _mlp_body(x_ref, w1t_ref, b1_ref, w2_ref, b2_ref, o_ref):
    h = jnp.dot(x_ref[...], w1t_ref[...], preferred_element_type=jnp.float32)
    h = jnp.maximum(h + b1_ref[...], 0.0)
    z = jnp.sum(h * w2_ref[...], axis=1, keepdims=True) + b2_ref[...]
    o_ref[...] = 1.0 / (1.0 + jnp.exp(-z))


def _mlp_tc(x, w1t, b1, w2, b2):
    BM = 1024
    nbatch = x.shape[0]
    grid = (nbatch // BM,)
    return pl.pallas_call(
        _mlp_body,
        grid=grid,
        in_specs=[
            pl.BlockSpec((BM, 2 * SKILL_DIM), lambda i: (i, 0)),
            pl.BlockSpec((2 * SKILL_DIM, HIDDEN_DIM), lambda i: (0, 0)),
            pl.BlockSpec((1, HIDDEN_DIM), lambda i: (0, 0)),
            pl.BlockSpec((1, HIDDEN_DIM), lambda i: (0, 0)),
            pl.BlockSpec((1, 1), lambda i: (0, 0)),
        ],
        out_specs=pl.BlockSpec((BM, 1), lambda i: (i, 0)),
        out_shape=jax.ShapeDtypeStruct((nbatch, 1), jnp.float32),
    )(x, w1t, b1, w2, b2)


def kernel(skill_ids_1, skill_ids_2, table, W1, b1, W2, b2):
    ids1 = skill_ids_1.astype(jnp.int32)
    ids2 = skill_ids_2.astype(jnp.int32)
    w1t = W1.T * (1.0 / LIST_LEN)
    b1r = b1.reshape(1, -1)
    b2r = b2.reshape(1, 1)
    h = BATCH // 2
    hc = h // CHUNK
    outs = []
    for s in range(2):
        i1 = ids1[s * h:(s + 1) * h].reshape(hc, UNIT_ROWS, 128)
        i2 = ids2[s * h:(s + 1) * h].reshape(hc, UNIT_ROWS, 128)
        pooled = _pooling_sc(table, i1, i2, h)
        outs.append(_mlp_tc(pooled, w1t, b1r, W2, b2r))
    return jnp.concatenate(outs, axis=0)


# speedup vs baseline: 1.4480x; 1.0994x over previous
import functools

import jax
import jax.numpy as jnp
from jax import lax
from jax.experimental import pallas as pl
from jax.experimental.pallas import tpu as pltpu
from jax.experimental.pallas import tpu_sc as plsc

BATCH = 16384
NUM_SKILLS = 100000
SKILL_DIM = 64
HIDDEN_DIM = 128
LIST_LEN = 20

NUM_CORES = 2
NUM_SUBCORES = 16
NW = NUM_CORES * NUM_SUBCORES

CHUNK = 32
IDS_PER_UNIT = CHUNK * LIST_LEN
UNIT_ROWS = IDS_PER_UNIT // 128
CHUNKS_PER_W = BATCH // (NW * CHUNK)
TOTAL_CHUNKS = BATCH // CHUNK


def _pooling_sc(table, idx1, idx2, nbatch):
    chunks_per_w = nbatch // (NW * CHUNK)
    mesh = plsc.VectorSubcoreMesh(
        core_axis_name="c", subcore_axis_name="s",
        num_cores=NUM_CORES, num_subcores=NUM_SUBCORES)

    @functools.partial(
        pl.kernel,
        out_type=jax.ShapeDtypeStruct((nbatch, 2 * SKILL_DIM), jnp.float32),
        mesh=mesh,
        scratch_types=[
            pltpu.VMEM((2, UNIT_ROWS, 128), jnp.int32),
            pltpu.VMEM((2, UNIT_ROWS, 128), jnp.int32),
            pltpu.VMEM((IDS_PER_UNIT, SKILL_DIM), jnp.bfloat16),
            pltpu.VMEM((IDS_PER_UNIT, SKILL_DIM), jnp.bfloat16),
            pltpu.VMEM((CHUNK, 2 * SKILL_DIM), jnp.float32),
            pltpu.SemaphoreType.DMA,
            pltpu.SemaphoreType.DMA,
            pltpu.SemaphoreType.DMA,
            pltpu.SemaphoreType.DMA,
        ],
        compiler_params=pltpu.CompilerParams(
            use_tc_tiling_on_sc=False, needs_layout_passes=False),
    )
    def k(table_hbm, idx1_hbm, idx2_hbm, out_hbm,
          idx_a, idx_b, rows_a, rows_b, out_v,
          sem_a, sem_b, isem_a, isem_b):
        wid = lax.axis_index("s") * NUM_CORES + lax.axis_index("c")
        base = wid * chunks_per_w

        def idx_copy(kk, src_hbm, idx_v, p, isem):
            pltpu.async_copy(src_hbm.at[kk], idx_v.at[p], isem)

        def fire(kk, src_hbm, idx_v, p, isem, rows_v, sem):
            pltpu.make_async_copy(src_hbm.at[kk], idx_v.at[p], isem).wait()
            for j in range(UNIT_ROWS):
                pltpu.async_copy(
                    table_hbm.at[idx_v.at[p, j]],
                    rows_v.at[pl.ds(j * 128, 128)], sem)

        def drain(idx_v, p, rows_v, sem):
            for j in range(UNIT_ROWS):
                pltpu.make_async_copy(
                    table_hbm.at[idx_v.at[p, j]],
                    rows_v.at[pl.ds(j * 128, 128)], sem).wait()

        def reduce(rows_v, half):
            @pl.loop(0, CHUNK)
            def _elem(i):
                rbase = i * LIST_LEN
                for d2 in range(SKILL_DIM // 32):
                    v = rows_v[rbase, pl.ds(d2 * 32, 32)]
                    acc_e, acc_o = plsc.unpack(
                        v, format=plsc.PackFormat.INTERLEAVED)
                    for r in range(1, LIST_LEN):
                        v = rows_v[rbase + r, pl.ds(d2 * 32, 32)]
                        e, o = plsc.unpack(
                            v, format=plsc.PackFormat.INTERLEAVED)
                        acc_e = acc_e + e
                        acc_o = acc_o + o
                    col = half * SKILL_DIM + d2 * 32
                    out_v[i, pl.ds(col, 16)] = acc_e
                    out_v[i, pl.ds(col + 16, 16)] = acc_o

        idx_copy(base, idx1_hbm, idx_a, 0, isem_a)
        idx_copy(base, idx2_hbm, idx_b, 0, isem_b)
        idx_copy(base + 1, idx1_hbm, idx_a, 1, isem_a)
        idx_copy(base + 1, idx2_hbm, idx_b, 1, isem_b)
        fire(base, idx1_hbm, idx_a, 0, isem_a, rows_a, sem_a)

        @pl.loop(0, chunks_per_w)
        def _chunk(c):
            kk = base + c
            p = c % 2
            fire(kk, idx2_hbm, idx_b, p, isem_b, rows_b, sem_b)
            drain(idx_a, p, rows_a, sem_a)

            @pl.when(c < chunks_per_w - 2)
            def _():
                idx_copy(kk + 2, idx1_hbm, idx_a, p, isem_a)

            reduce(rows_a, 0)

            @pl.when(c < chunks_per_w - 1)
            def _():
                fire(kk + 1, idx1_hbm, idx_a, 1 - p, isem_a, rows_a, sem_a)

            drain(idx_b, p, rows_b, sem_b)

            @pl.when(c < chunks_per_w - 2)
            def _():
                idx_copy(kk + 2, idx2_hbm, idx_b, p, isem_b)

            reduce(rows_b, 1)
            pltpu.sync_copy(out_v, out_hbm.at[pl.ds(kk * CHUNK, CHUNK)])

    return k(table, idx1, idx2)


def _mlp_body(x_ref, w1t_ref, b1_ref, w2_ref, b2_ref, o_ref):
    h = jnp.dot(x_ref[...], w1t_ref[...], preferred_element_type=jnp.float32)
    h = jnp.maximum(h + b1_ref[...], 0.0)
    z = jnp.sum(h * w2_ref[...], axis=1, keepdims=True) + b2_ref[...]
    o_ref[...] = 1.0 / (1.0 + jnp.exp(-z))


def _mlp_tc(x, w1t, b1, w2, b2):
    BM = 1024
    nbatch = x.shape[0]
    grid = (nbatch // BM,)
    return pl.pallas_call(
        _mlp_body,
        grid=grid,
        in_specs=[
            pl.BlockSpec((BM, 2 * SKILL_DIM), lambda i: (i, 0)),
            pl.BlockSpec((2 * SKILL_DIM, HIDDEN_DIM), lambda i: (0, 0)),
            pl.BlockSpec((1, HIDDEN_DIM), lambda i: (0, 0)),
            pl.BlockSpec((1, HIDDEN_DIM), lambda i: (0, 0)),
            pl.BlockSpec((1, 1), lambda i: (0, 0)),
        ],
        out_specs=pl.BlockSpec((BM, 1), lambda i: (i, 0)),
        out_shape=jax.ShapeDtypeStruct((nbatch, 1), jnp.float32),
    )(x, w1t, b1, w2, b2)


def kernel(skill_ids_1, skill_ids_2, table, W1, b1, W2, b2):
    ids1 = skill_ids_1.astype(jnp.int32)
    ids2 = skill_ids_2.astype(jnp.int32)
    table_bf = table.astype(jnp.bfloat16)
    w1t = W1.T * (1.0 / LIST_LEN)
    perm = []
    for hh in range(2):
        for d2 in range(SKILL_DIM // 32):
            for sub in range(2):
                for s in range(16):
                    perm.append(hh * SKILL_DIM + d2 * 32 + 2 * s + sub)
    w1t = w1t[jnp.array(perm), :]
    b1r = b1.reshape(1, -1)
    b2r = b2.reshape(1, 1)
    h = BATCH // 2
    hc = h // CHUNK
    outs = []
    for s in range(2):
        i1 = ids1[s * h:(s + 1) * h].reshape(hc, UNIT_ROWS, 128)
        i2 = ids2[s * h:(s + 1) * h].reshape(hc, UNIT_ROWS, 128)
        pooled = _pooling_sc(table_bf, i1, i2, h)
        outs.append(_mlp_tc(pooled, w1t, b1r, W2, b2r))
    return jnp.concatenate(outs, axis=0)
